# baseline (speedup 1.0000x reference)
import functools

import jax
import jax.numpy as jnp
from jax import lax
from jax.experimental import pallas as pl
from jax.experimental.pallas import tpu as pltpu
from jax.experimental.pallas import tpu_sc as plsc

N = 4096
DIM_IN = 384
DIM_H = 64
TOPK = 6
ROW_BLK = 256
NUM_BLKS = N // ROW_BLK
NEG = -1e30

_NUM_SC_CORES = 2
_NW = 32
_B_PER_W = (TOPK * N) // _NW
_CHUNK = 128
_N_CHUNKS = _B_PER_W // _CHUNK


def _fc1_body(feats_ref, w1t_ref, b1_ref, out_ref):
    v = jnp.dot(feats_ref[...], w1t_ref[...],
                preferred_element_type=jnp.float32) + b1_ref[...]
    out_ref[...] = jnp.where(v >= 0, v, 0.01 * v)


def _leaky(v):
    return jnp.where(v >= 0, v, 0.01 * v)


def _mean_rows(x):
    xr = x.reshape(N // 128, 16, 8, DIM_H)
    acc = jnp.zeros((16, 8, DIM_H), jnp.float32)
    for k in range(N // 128):
        acc = acc + xr[k]
    s = acc[0]
    for u in range(1, 16):
        s = s + acc[u]
    s = s[:4] + s[4:]
    s = s[:2] + s[2:]
    s = s[0:1] + s[1:2]
    return s * (1.0 / N)


def _front_body(x_ref, wht_ref, bh_ref, wtt_ref, bt_ref,
                eh_o, et_o, w_o, idx_o, eh_s, et_s):
    i = pl.program_id(0)

    @pl.when(i == 0)
    def _prologue():
        xm = _mean_rows(x_ref[...])
        x = (x_ref[...] + xm) * 0.5
        eh_s[...] = jnp.dot(x, wht_ref[...],
                            preferred_element_type=jnp.float32) + bh_ref[...]
        et_s[...] = jnp.dot(x, wtt_ref[...],
                            preferred_element_type=jnp.float32) + bt_ref[...]
        eh_o[...] = eh_s[...]
        et_o[...] = et_s[...]

    rows = pl.ds(i * ROW_BLK, ROW_BLK)
    e_h = eh_s[rows, :]
    e_t = et_s[...]
    scale = DIM_H ** (-0.5)
    logits = lax.dot_general(e_h * scale, e_t,
                             (((1,), (1,)), ((), ())),
                             preferred_element_type=jnp.float32)

    iota = lax.broadcasted_iota(jnp.int32, (ROW_BLK, N), 1)
    ws = []
    idxs = []
    for _ in range(TOPK):
        m = jnp.max(logits, axis=1, keepdims=True)
        idx = jnp.min(jnp.where(logits == m, iota, N), axis=1,
                      keepdims=True)
        ws.append(m)
        idxs.append(idx)
        logits = jnp.where(iota == idx, NEG, logits)
    w_o[...] = jnp.concatenate(ws, axis=1)
    idx_o[...] = jnp.concatenate(idxs, axis=1)


def _sc_gather(table, idx_flat):
    mesh = plsc.VectorSubcoreMesh(core_axis_name="c", subcore_axis_name="s")

    @functools.partial(
        pl.kernel, mesh=mesh,
        compiler_params=pltpu.CompilerParams(use_tc_tiling_on_sc=False),
        out_type=jax.ShapeDtypeStruct((TOPK * N, DIM_H), jnp.float32),
        scratch_types=[
            pltpu.VMEM((_B_PER_W,), jnp.int32),
            pltpu.VMEM((_B_PER_W, DIM_H), jnp.float32),
            pltpu.SemaphoreType.DMA,
        ],
    )
    def gather_kernel(table_hbm, idx_hbm, out_hbm, idx_v, rows_v, sem):
        wid = lax.axis_index("s") * _NUM_SC_CORES + lax.axis_index("c")
        base = wid * _B_PER_W
        pltpu.sync_copy(idx_hbm.at[pl.ds(base, _B_PER_W)], idx_v)
        copies = []
        for c in range(_N_CHUNKS):
            copies.append(pltpu.async_copy(
                table_hbm.at[idx_v.at[pl.ds(c * _CHUNK, _CHUNK)]],
                rows_v.at[pl.ds(c * _CHUNK, _CHUNK), :], sem))
        for cp in copies:
            cp.wait()
        pltpu.sync_copy(rows_v, out_hbm.at[pl.ds(base, _B_PER_W)])

    return gather_kernel(table, idx_flat)


def _tail_body(eh_ref, w_ref, nb_ref,
               wl1t_ref, bl1_ref, wl2t_ref, bl2_ref,
               gamma_ref, beta_ref, wct_ref, bc_ref,
               out_ref, acc_s):
    i = pl.program_id(0)

    @pl.when(i == 0)
    def _init():
        acc_s[...] = jnp.zeros_like(acc_s)

    e_h = eh_ref[...]
    w = w_ref[...]
    nbs = [nb_ref[j] for j in range(TOPK)]

    exps = jnp.exp(w - w[:, 0:1])
    z = jnp.sum(exps, axis=1, keepdims=True)
    ps = [exps[:, j:j + 1] / z for j in range(TOPK)]

    kas = []
    for p, nb in zip(ps, nbs):
        eh_r = p * nb + (1.0 - p) * e_h
        gate = jnp.tanh(e_h + eh_r)
        kas.append(jnp.sum(nb, axis=1, keepdims=True) *
                   jnp.sum(gate, axis=1, keepdims=True))
    ka_max = functools.reduce(jnp.maximum, kas)
    qs = [jnp.exp(ka - ka_max) for ka in kas]
    qz = functools.reduce(jnp.add, qs)
    e_nh = functools.reduce(
        jnp.add, [(q / qz) * nb for q, nb in zip(qs, nbs)])

    sum_emb = _leaky(jnp.dot(e_h + e_nh, wl1t_ref[...],
                             preferred_element_type=jnp.float32) + bl1_ref[...])
    bi_emb = _leaky(jnp.dot(e_h * e_nh, wl2t_ref[...],
                            preferred_element_type=jnp.float32) + bl2_ref[...])
    emb = sum_emb + bi_emb
    acc_s[...] += jnp.sum(emb, axis=0, keepdims=True)

    @pl.when(i == NUM_BLKS - 1)
    def _epilogue():
        h = acc_s[...] / N
        mu = jnp.mean(h, axis=1, keepdims=True)
        var = jnp.mean((h - mu) ** 2, axis=1, keepdims=True)
        hn = (h - mu) / jnp.sqrt(var + 1e-5) * gamma_ref[...] + beta_ref[...]
        out_ref[...] = jnp.dot(hn, wct_ref[...],
                               preferred_element_type=jnp.float32) + bc_ref[...]


def kernel(feats, W1, b1, Wh, bh, Wt, bt, Wl1, bl1, Wl2, bl2, gamma, beta, Wc, bc):
    f2 = feats.reshape(N, DIM_IN)

    x_raw = pl.pallas_call(
        _fc1_body,
        grid=(NUM_BLKS,),
        in_specs=[
            pl.BlockSpec((ROW_BLK, DIM_IN), lambda i: (i, 0)),
            pl.BlockSpec((DIM_IN, DIM_H), lambda i: (0, 0)),
            pl.BlockSpec((1, DIM_H), lambda i: (0, 0)),
        ],
        out_specs=pl.BlockSpec((ROW_BLK, DIM_H), lambda i: (i, 0)),
        out_shape=jax.ShapeDtypeStruct((N, DIM_H), jnp.float32),
    )(f2, W1.T, b1.reshape(1, DIM_H))

    full = lambda s: pl.BlockSpec(s, lambda i: tuple(0 for _ in s))
    e_h, e_t, w, idx = pl.pallas_call(
        _front_body,
        grid=(NUM_BLKS,),
        in_specs=[
            full((N, DIM_H)),
            full((DIM_H, DIM_H)), full((1, DIM_H)),
            full((DIM_H, DIM_H)), full((1, DIM_H)),
        ],
        out_specs=[full((N, DIM_H)), full((N, DIM_H)),
                   pl.BlockSpec((ROW_BLK, TOPK), lambda i: (i, 0)),
                   pl.BlockSpec((ROW_BLK, TOPK), lambda i: (i, 0))],
        out_shape=[jax.ShapeDtypeStruct((N, DIM_H), jnp.float32),
                   jax.ShapeDtypeStruct((N, DIM_H), jnp.float32),
                   jax.ShapeDtypeStruct((N, TOPK), jnp.float32),
                   jax.ShapeDtypeStruct((N, TOPK), jnp.int32)],
        scratch_shapes=[
            pltpu.VMEM((N, DIM_H), jnp.float32),
            pltpu.VMEM((N, DIM_H), jnp.float32),
        ],
    )(x_raw, Wh.T, bh.reshape(1, DIM_H), Wt.T, bt.reshape(1, DIM_H))

    idx_flat = idx.T.reshape(TOPK * N)
    nb = _sc_gather(e_t, idx_flat)
    nb3 = nb.reshape(TOPK, N, DIM_H)

    out = pl.pallas_call(
        _tail_body,
        grid=(NUM_BLKS,),
        in_specs=[
            pl.BlockSpec((ROW_BLK, DIM_H), lambda i: (i, 0)),
            pl.BlockSpec((ROW_BLK, TOPK), lambda i: (i, 0)),
            pl.BlockSpec((TOPK, ROW_BLK, DIM_H), lambda i: (0, i, 0)),
            full((DIM_H, DIM_H)), full((1, DIM_H)),
            full((DIM_H, DIM_H)), full((1, DIM_H)),
            full((1, DIM_H)), full((1, DIM_H)),
            full((DIM_H, 2)), full((1, 2)),
        ],
        out_specs=full((1, 2)),
        out_shape=jax.ShapeDtypeStruct((1, 2), jnp.float32),
        scratch_shapes=[
            pltpu.VMEM((1, DIM_H), jnp.float32),
        ],
    )(e_h, w, nb3,
      Wl1.T, bl1.reshape(1, DIM_H), Wl2.T, bl2.reshape(1, DIM_H),
      gamma.reshape(1, DIM_H), beta.reshape(1, DIM_H),
      Wc.T, bc.reshape(1, 2))
    return out


# speedup vs baseline: 1.0456x; 1.0456x over previous
import functools

import jax
import jax.numpy as jnp
from jax import lax
from jax.experimental import pallas as pl
from jax.experimental.pallas import tpu as pltpu
from jax.experimental.pallas import tpu_sc as plsc

N = 4096
DIM_IN = 384
DIM_H = 64
TOPK = 6
ROW_BLK = 256
NUM_BLKS = N // ROW_BLK
NEG = -1e30

_NUM_SC_CORES = 2
_NW = 32
_B_PER_W = (TOPK * N) // _NW
_CHUNK = 128
_N_CHUNKS = _B_PER_W // _CHUNK


def _leaky(v):
    return jnp.where(v >= 0, v, 0.01 * v)


def _mean_rows(x):
    xr = x.reshape(N // 128, 16, 8, DIM_H)
    acc = jnp.zeros((16, 8, DIM_H), jnp.float32)
    for k in range(N // 128):
        acc = acc + xr[k]
    s = acc[0]
    for u in range(1, 16):
        s = s + acc[u]
    s = s[:4] + s[4:]
    s = s[:2] + s[2:]
    s = s[0:1] + s[1:2]
    return s * (1.0 / N)


def _front_body(feats_ref, w1t_ref, b1_ref, wht_ref, bh_ref, wtt_ref, bt_ref,
                eh_o, et_o, w_o, idx_o, xraw_s, eh_s, et_s):
    i = pl.program_id(0)

    @pl.when(i == 0)
    def _prologue():
        for b in range(NUM_BLKS):
            rows_b = pl.ds(b * ROW_BLK, ROW_BLK)
            v = jnp.dot(feats_ref[rows_b, :], w1t_ref[...],
                        preferred_element_type=jnp.float32) + b1_ref[...]
            xraw_s[rows_b, :] = _leaky(v)
        xm = _mean_rows(xraw_s[...])
        x = (xraw_s[...] + xm) * 0.5
        eh_s[...] = jnp.dot(x, wht_ref[...],
                            preferred_element_type=jnp.float32) + bh_ref[...]
        et_s[...] = jnp.dot(x, wtt_ref[...],
                            preferred_element_type=jnp.float32) + bt_ref[...]
        eh_o[...] = eh_s[...]
        et_o[...] = et_s[...]

    rows = pl.ds(i * ROW_BLK, ROW_BLK)
    e_h = eh_s[rows, :]
    e_t = et_s[...]
    scale = DIM_H ** (-0.5)
    logits = lax.dot_general(e_h * scale, e_t,
                             (((1,), (1,)), ((), ())),
                             preferred_element_type=jnp.float32)

    iota = lax.broadcasted_iota(jnp.int32, (ROW_BLK, N), 1)
    ws = []
    idxs = []
    for _ in range(TOPK):
        m = jnp.max(logits, axis=1, keepdims=True)
        idx = jnp.min(jnp.where(logits == m, iota, N), axis=1,
                      keepdims=True)
        ws.append(m)
        idxs.append(idx)
        logits = jnp.where(iota == idx, NEG, logits)
    w_o[...] = jnp.concatenate(ws, axis=1)
    idx_o[...] = jnp.concatenate(idxs, axis=1)


def _sc_gather(table, idx_flat):
    mesh = plsc.VectorSubcoreMesh(core_axis_name="c", subcore_axis_name="s")

    @functools.partial(
        pl.kernel, mesh=mesh,
        compiler_params=pltpu.CompilerParams(use_tc_tiling_on_sc=False),
        out_type=jax.ShapeDtypeStruct((TOPK * N, DIM_H), jnp.float32),
        scratch_types=[
            pltpu.VMEM((_B_PER_W,), jnp.int32),
            pltpu.VMEM((_B_PER_W, DIM_H), jnp.float32),
            pltpu.SemaphoreType.DMA,
        ],
    )
    def gather_kernel(table_hbm, idx_hbm, out_hbm, idx_v, rows_v, sem):
        wid = lax.axis_index("s") * _NUM_SC_CORES + lax.axis_index("c")
        base = wid * _B_PER_W
        pltpu.sync_copy(idx_hbm.at[pl.ds(base, _B_PER_W)], idx_v)
        copies = []
        for c in range(_N_CHUNKS):
            copies.append(pltpu.async_copy(
                table_hbm.at[idx_v.at[pl.ds(c * _CHUNK, _CHUNK)]],
                rows_v.at[pl.ds(c * _CHUNK, _CHUNK), :], sem))
        for cp in copies:
            cp.wait()
        pltpu.sync_copy(rows_v, out_hbm.at[pl.ds(base, _B_PER_W)])

    return gather_kernel(table, idx_flat)


def _tail_body(eh_ref, w_ref, nb_ref,
               wl1t_ref, bl1_ref, wl2t_ref, bl2_ref,
               gamma_ref, beta_ref, wct_ref, bc_ref,
               out_ref, acc_s):
    i = pl.program_id(0)

    @pl.when(i == 0)
    def _init():
        acc_s[...] = jnp.zeros_like(acc_s)

    e_h = eh_ref[...]
    w = w_ref[...]
    nbs = [nb_ref[j] for j in range(TOPK)]

    exps = jnp.exp(w - w[:, 0:1])
    z = jnp.sum(exps, axis=1, keepdims=True)
    ps = [exps[:, j:j + 1] / z for j in range(TOPK)]

    kas = []
    for p, nb in zip(ps, nbs):
        eh_r = p * nb + (1.0 - p) * e_h
        gate = jnp.tanh(e_h + eh_r)
        kas.append(jnp.sum(nb, axis=1, keepdims=True) *
                   jnp.sum(gate, axis=1, keepdims=True))
    ka_max = functools.reduce(jnp.maximum, kas)
    qs = [jnp.exp(ka - ka_max) for ka in kas]
    qz = functools.reduce(jnp.add, qs)
    e_nh = functools.reduce(
        jnp.add, [(q / qz) * nb for q, nb in zip(qs, nbs)])

    sum_emb = _leaky(jnp.dot(e_h + e_nh, wl1t_ref[...],
                             preferred_element_type=jnp.float32) + bl1_ref[...])
    bi_emb = _leaky(jnp.dot(e_h * e_nh, wl2t_ref[...],
                            preferred_element_type=jnp.float32) + bl2_ref[...])
    emb = sum_emb + bi_emb
    acc_s[...] += jnp.sum(emb, axis=0, keepdims=True)

    @pl.when(i == NUM_BLKS - 1)
    def _epilogue():
        h = acc_s[...] / N
        mu = jnp.mean(h, axis=1, keepdims=True)
        var = jnp.mean((h - mu) ** 2, axis=1, keepdims=True)
        hn = (h - mu) / jnp.sqrt(var + 1e-5) * gamma_ref[...] + beta_ref[...]
        out_ref[...] = jnp.dot(hn, wct_ref[...],
                               preferred_element_type=jnp.float32) + bc_ref[...]


def kernel(feats, W1, b1, Wh, bh, Wt, bt, Wl1, bl1, Wl2, bl2, gamma, beta, Wc, bc):
    f2 = feats.reshape(N, DIM_IN)

    full = lambda s: pl.BlockSpec(s, lambda i: tuple(0 for _ in s))
    e_h, e_t, w, idx = pl.pallas_call(
        _front_body,
        grid=(NUM_BLKS,),
        in_specs=[
            full((N, DIM_IN)),
            full((DIM_IN, DIM_H)), full((1, DIM_H)),
            full((DIM_H, DIM_H)), full((1, DIM_H)),
            full((DIM_H, DIM_H)), full((1, DIM_H)),
        ],
        out_specs=[full((N, DIM_H)), full((N, DIM_H)),
                   pl.BlockSpec((ROW_BLK, TOPK), lambda i: (i, 0)),
                   pl.BlockSpec((ROW_BLK, TOPK), lambda i: (i, 0))],
        out_shape=[jax.ShapeDtypeStruct((N, DIM_H), jnp.float32),
                   jax.ShapeDtypeStruct((N, DIM_H), jnp.float32),
                   jax.ShapeDtypeStruct((N, TOPK), jnp.float32),
                   jax.ShapeDtypeStruct((N, TOPK), jnp.int32)],
        scratch_shapes=[
            pltpu.VMEM((N, DIM_H), jnp.float32),
            pltpu.VMEM((N, DIM_H), jnp.float32),
            pltpu.VMEM((N, DIM_H), jnp.float32),
        ],
    )(f2, W1.T, b1.reshape(1, DIM_H),
      Wh.T, bh.reshape(1, DIM_H), Wt.T, bt.reshape(1, DIM_H))

    idx_flat = idx.T.reshape(TOPK * N)
    nb = _sc_gather(e_t, idx_flat)
    nb3 = nb.reshape(TOPK, N, DIM_H)

    out = pl.pallas_call(
        _tail_body,
        grid=(NUM_BLKS,),
        in_specs=[
            pl.BlockSpec((ROW_BLK, DIM_H), lambda i: (i, 0)),
            pl.BlockSpec((ROW_BLK, TOPK), lambda i: (i, 0)),
            pl.BlockSpec((TOPK, ROW_BLK, DIM_H), lambda i: (0, i, 0)),
            full((DIM_H, DIM_H)), full((1, DIM_H)),
            full((DIM_H, DIM_H)), full((1, DIM_H)),
            full((1, DIM_H)), full((1, DIM_H)),
            full((DIM_H, 2)), full((1, 2)),
        ],
        out_specs=full((1, 2)),
        out_shape=jax.ShapeDtypeStruct((1, 2), jnp.float32),
        scratch_shapes=[
            pltpu.VMEM((1, DIM_H), jnp.float32),
        ],
    )(e_h, w, nb3,
      Wl1.T, bl1.reshape(1, DIM_H), Wl2.T, bl2.reshape(1, DIM_H),
      gamma.reshape(1, DIM_H), beta.reshape(1, DIM_H),
      Wc.T, bc.reshape(1, 2))
    return out
